# baseline (speedup 1.0000x reference)
import functools

import jax
import jax.numpy as jnp
from jax import lax
from jax.experimental import pallas as pl
from jax.experimental.pallas import tpu as pltpu
from jax.experimental.pallas import tpu_sc as plsc

NUM_SEG = 10000
N_ROWS = 320000
D = 128
NC, NS = 2, 16
SEG_PER_SC = NUM_SEG // NC
ACC_ROWS = 5008
CHUNK = 80
NCHUNKS_TOT = N_ROWS // CHUNK
MAXCH_T = NCHUNKS_TOT // NS
NBUF = 5
PIECE = 40
NPIECE = SEG_PER_SC // PIECE

_mesh = plsc.VectorSubcoreMesh(core_axis_name="c", subcore_axis_name="s")


@functools.partial(
    pl.kernel,
    out_type=jax.ShapeDtypeStruct((NUM_SEG, D), jnp.float32),
    mesh=_mesh,
    scratch_types=[
        pltpu.VMEM((16,), jnp.int32),
        pltpu.VMEM((CHUNK * MAXCH_T,), jnp.int32),
        pltpu.VMEM((NBUF, CHUNK), jnp.int32),
        [pltpu.VMEM((CHUNK, D), jnp.float32) for _ in range(NBUF)],
        pltpu.VMEM((PIECE, D), jnp.float32),
        pltpu.VMEM_SHARED((ACC_ROWS, D), jnp.float32),
        [pltpu.SemaphoreType.DMA for _ in range(NBUF)],
        [pltpu.SemaphoreType.DMA for _ in range(NBUF)],
    ],
)
def _sc_segment_sum(values_hbm, ids_hbm, params_hbm, zeros_hbm, out_hbm,
                    params_v, idraw, ids2d, bufs, stage, acc, lsems, ssems):
    c = lax.axis_index("c")
    s = lax.axis_index("s")

    pltpu.sync_copy(params_hbm, params_v)
    pvec = params_v[...]
    n0 = pvec[0]
    n1 = pvec[1]
    g0 = pvec[2]
    g1 = pvec[3]
    is0 = c == 0
    n = lax.select(is0, n0, n1)
    ngroups = lax.select(is0, g0, g1)
    startchunk = lax.select(is0, s * n0, NCHUNKS_TOT - (s + 1) * n1)

    win = CHUNK * MAXCH_T
    idoff = lax.select(is0, startchunk * CHUNK,
                       (startchunk + n1) * CHUNK - win)
    pltpu.sync_copy(ids_hbm.at[pl.ds(idoff, win)], idraw)
    boff = lax.select(is0, 0, win - n1 * CHUNK)

    for k in range(-(-NPIECE // NS)):
        p = k * NS + s

        @pl.when(p < NPIECE)
        def _():
            pltpu.sync_copy(zeros_hbm, acc.at[pl.ds(p * PIECE, PIECE)])
    plsc.subcore_barrier()

    base = (c * SEG_PER_SC).astype(jnp.int32)
    garbage = jnp.full((16,), SEG_PER_SC, jnp.int32)
    nchunks = ngroups * NBUF

    def load(jj, b):
        gch = jnp.minimum(startchunk + jj, NCHUNKS_TOT - 1)
        pltpu.async_copy(
            values_hbm.at[pl.ds(gch * CHUNK, CHUNK)], bufs[b], lsems[b])

    load(0, 0)
    load(1, 1)

    def chunk_group(g, carry):
        for u in range(NBUF):
            jj = NBUF * g + u

            @pl.when(jj >= 3)
            def _():
                pltpu.make_async_copy(
                    values_hbm.at[pl.ds(0, CHUNK)],
                    bufs[(u - 3) % NBUF], ssems[(u - 3) % NBUF]).wait()

            @pl.when(jj + 2 < nchunks)
            def _():
                load(jj + 2, (u + 2) % NBUF)

            limit = lax.select(jj < n, jnp.int32(SEG_PER_SC), jnp.int32(0))
            offc = jnp.minimum(boff + jj * CHUNK, win - CHUNK)
            for k in range(CHUNK // 16):
                v = idraw[pl.ds(offc + 16 * k, 16)] - base
                ok = (v >= 0) & (v < limit)
                ids2d[u, pl.ds(16 * k, 16)] = jnp.where(ok, v, garbage)

            pltpu.make_async_copy(
                values_hbm.at[pl.ds(0, CHUNK)], bufs[u], lsems[u]).wait()
            pltpu.async_copy(bufs[u], acc.at[ids2d.at[u]], ssems[u], add=True)
        return carry

    lax.fori_loop(0, ngroups, chunk_group, 0, unroll=False)
    for b in (NBUF - 3, NBUF - 2, NBUF - 1):
        pltpu.make_async_copy(
            values_hbm.at[pl.ds(0, CHUNK)], bufs[b], ssems[b]).wait()
    plsc.subcore_barrier()

    for k in range(-(-NPIECE // NS)):
        p = k * NS + s

        @pl.when(p < NPIECE)
        def _():
            pltpu.sync_copy(
                acc.at[pl.ds(p * PIECE, PIECE)],
                out_hbm.at[pl.ds(c * SEG_PER_SC + p * PIECE, PIECE)])


@jax.jit
def kernel(values, segment_ids):
    ids = segment_ids.astype(jnp.int32)
    split = jnp.sum((ids < SEG_PER_SC).astype(jnp.int32)).astype(jnp.int32)
    rows_per_round = NS * CHUNK
    n0 = (split + rows_per_round - 1) // rows_per_round
    n1 = (N_ROWS - split + rows_per_round - 1) // rows_per_round
    g0 = jnp.maximum(-(-n0 // NBUF), 1)
    g1 = jnp.maximum(-(-n1 // NBUF), 1)
    params = jnp.zeros((16,), jnp.int32).at[0].set(n0).at[1].set(n1)
    params = params.at[2].set(g0).at[3].set(g1)
    zeros = jnp.zeros((PIECE, D), jnp.float32)
    return _sc_segment_sum(values, ids, params, zeros)


# speedup vs baseline: 1.0683x; 1.0683x over previous
import functools

import jax
import jax.numpy as jnp
from jax import lax
from jax.experimental import pallas as pl
from jax.experimental.pallas import tpu as pltpu
from jax.experimental.pallas import tpu_sc as plsc

NUM_SEG = 10000
N_ROWS = 320000
D = 128
NC, NS = 2, 16
SEG_PER_SC = NUM_SEG // NC
ACC_ROWS = 5008
CHUNK = 80
NCHUNKS_TOT = N_ROWS // CHUNK
MAXCH_T = NCHUNKS_TOT // NS
NBUF = 5
PIECE = 40
NPIECE = SEG_PER_SC // PIECE

_mesh = plsc.VectorSubcoreMesh(core_axis_name="c", subcore_axis_name="s")


@functools.partial(
    pl.kernel,
    out_type=jax.ShapeDtypeStruct((NUM_SEG, D), jnp.float32),
    mesh=_mesh,
    scratch_types=[
        pltpu.VMEM((16,), jnp.int32),
        pltpu.VMEM((CHUNK * MAXCH_T,), jnp.int32),
        pltpu.VMEM((NBUF, CHUNK), jnp.int32),
        [pltpu.VMEM((CHUNK, D), jnp.float32) for _ in range(NBUF)],
        pltpu.VMEM((PIECE, D), jnp.float32),
        pltpu.VMEM_SHARED((ACC_ROWS, D), jnp.float32),
        [pltpu.SemaphoreType.DMA for _ in range(NBUF)],
        [pltpu.SemaphoreType.DMA for _ in range(NBUF)],
    ],
)
def _sc_segment_sum(values_hbm, ids_hbm, params_hbm, zeros_hbm, out_hbm,
                    params_v, idraw, ids2d, bufs, stage, acc, lsems, ssems):
    c = lax.axis_index("c")
    s = lax.axis_index("s")

    pltpu.sync_copy(params_hbm, params_v)
    pvec = params_v[...]
    n0 = pvec[0]
    n1 = pvec[1]
    g0 = pvec[2]
    g1 = pvec[3]
    is0 = c == 0
    n = lax.select(is0, n0, n1)
    ngroups = lax.select(is0, g0, g1)
    startchunk = lax.select(is0, s * n0, NCHUNKS_TOT - (s + 1) * n1)

    win = CHUNK * MAXCH_T
    idoff = lax.select(is0, startchunk * CHUNK,
                       (startchunk + n1) * CHUNK - win)
    pltpu.sync_copy(ids_hbm.at[pl.ds(idoff, win)], idraw)
    boff = lax.select(is0, 0, win - n1 * CHUNK)

    pltpu.sync_copy(zeros_hbm, stage)
    for k in range(-(-NPIECE // NS)):
        p = k * NS + s

        @pl.when(p < NPIECE)
        def _():
            pltpu.sync_copy(stage, acc.at[pl.ds(p * PIECE, PIECE)])
    plsc.subcore_barrier()

    base = (c * SEG_PER_SC).astype(jnp.int32)
    garbage = jnp.full((16,), SEG_PER_SC, jnp.int32)
    nchunks = ngroups * NBUF

    def load(jj, b):
        gch = jnp.minimum(startchunk + jj, NCHUNKS_TOT - 1)
        pltpu.async_copy(
            values_hbm.at[pl.ds(gch * CHUNK, CHUNK)], bufs[b], lsems[b])

    load(0, 0)
    load(1, 1)

    def chunk_group(g, carry):
        for u in range(NBUF):
            jj = NBUF * g + u

            @pl.when(jj >= 3)
            def _():
                pltpu.make_async_copy(
                    values_hbm.at[pl.ds(0, CHUNK)],
                    bufs[(u - 3) % NBUF], ssems[(u - 3) % NBUF]).wait()

            @pl.when(jj + 2 < nchunks)
            def _():
                load(jj + 2, (u + 2) % NBUF)

            limit = lax.select(jj < n, jnp.int32(SEG_PER_SC), jnp.int32(0))
            offc = jnp.minimum(boff + jj * CHUNK, win - CHUNK)
            for k in range(CHUNK // 16):
                v = idraw[pl.ds(offc + 16 * k, 16)] - base
                ok = (v >= 0) & (v < limit)
                ids2d[u, pl.ds(16 * k, 16)] = jnp.where(ok, v, garbage)

            pltpu.make_async_copy(
                values_hbm.at[pl.ds(0, CHUNK)], bufs[u], lsems[u]).wait()
            pltpu.async_copy(bufs[u], acc.at[ids2d.at[u]], ssems[u], add=True)
        return carry

    lax.fori_loop(0, ngroups, chunk_group, 0, unroll=False)
    for b in (NBUF - 3, NBUF - 2, NBUF - 1):
        pltpu.make_async_copy(
            values_hbm.at[pl.ds(0, CHUNK)], bufs[b], ssems[b]).wait()
    plsc.subcore_barrier()

    for k in range(-(-NPIECE // NS)):
        p = k * NS + s

        @pl.when(p < NPIECE)
        def _():
            pltpu.sync_copy(acc.at[pl.ds(p * PIECE, PIECE)], stage)
            pltpu.sync_copy(
                stage, out_hbm.at[pl.ds(c * SEG_PER_SC + p * PIECE, PIECE)])


@jax.jit
def kernel(values, segment_ids):
    ids = segment_ids.astype(jnp.int32)
    split = jnp.sum((ids < SEG_PER_SC).astype(jnp.int32)).astype(jnp.int32)
    rows_per_round = NS * CHUNK
    n0 = (split + rows_per_round - 1) // rows_per_round
    n1 = (N_ROWS - split + rows_per_round - 1) // rows_per_round
    g0 = jnp.maximum(-(-n0 // NBUF), 1)
    g1 = jnp.maximum(-(-n1 // NBUF), 1)
    params = jnp.zeros((16,), jnp.int32).at[0].set(n0).at[1].set(n1)
    params = params.at[2].set(g0).at[3].set(g1)
    zeros = jnp.zeros((PIECE, D), jnp.float32)
    return _sc_segment_sum(values, ids, params, zeros)
